# baseline (speedup 1.0000x reference)
import functools

import jax
import jax.numpy as jnp
from jax import lax
from jax.experimental import pallas as pl
from jax.experimental.pallas import tpu as pltpu
from jax.experimental.pallas import tpu_sc as plsc

N_NODE = 100000
DIM = 32
NF = 4
NL = 2
B = 1024
K = 32
SEQ = 128
H = 768

NC = 2
NS = 16
NW = NC * NS

N_REAL = 2 * B + 8 * B * K
CHUNK = 128
CH_PER_W = 66
N_PAD = NW * CH_PER_W * CHUNK

ROWS_PER_W = B // NW
HALF = SEQ // 2


def _node_gather_body(node_hbm, idx_hbm, gout_hbm, idx_v, rows_v, out_v, sem0, sem1):
    wid = lax.axis_index("s") * NC + lax.axis_index("c")
    sems = (sem0, sem1)
    pltpu.sync_copy(idx_hbm.at[wid], idx_v)
    pltpu.async_copy(node_hbm.at[idx_v.at[0]], rows_v.at[0], sems[0])
    rnd = jnp.full((16,), 32768, jnp.int32)
    mask_hi = jnp.full((16,), -65536, jnp.int32)
    sh16 = jnp.full((16,), 16, jnp.int32)

    def pair(g, carry):
        for b in range(2):
            c = 2 * g + b
            pltpu.make_async_copy(node_hbm.at[idx_v.at[c]], rows_v.at[b], sems[b]).wait()

            @pl.when(c + 1 < CH_PER_W)
            def _():
                pltpu.async_copy(node_hbm.at[idx_v.at[c + 1]], rows_v.at[1 - b],
                                 sems[1 - b])

            def packrow(r, cc, _b=b):
                for q in range(4):
                    lo = lax.bitcast_convert_type(
                        rows_v[_b, r, pl.ds(q * 16, 16)], jnp.int32)
                    hi = lax.bitcast_convert_type(
                        rows_v[_b, r, pl.ds(64 + q * 16, 16)], jnp.int32)
                    lo = lax.shift_right_logical(lo + rnd, sh16)
                    hi = lax.bitwise_and(hi + rnd, mask_hi)
                    out_v[_b, r, pl.ds(q * 16, 16)] = lax.bitwise_or(lo, hi)
                return cc
            lax.fori_loop(0, CHUNK, packrow, 0)

            base = wid * (CH_PER_W * CHUNK) + c * CHUNK
            pltpu.sync_copy(out_v.at[b], gout_hbm.at[pl.ds(base, CHUNK)])
        return carry

    lax.fori_loop(0, CH_PER_W // 2, pair, 0)


def _lm_pool_body(tok_hbm, ids_hbm, psum_hbm, ids_v, rows_v, acc_v, sem0, sem1):
    wid = lax.axis_index("s") * NC + lax.axis_index("c")
    sems = (sem0, sem1)
    pltpu.sync_copy(ids_hbm.at[wid], ids_v)

    pltpu.async_copy(tok_hbm.at[ids_v.at[0]], rows_v.at[0], sems[0])
    pltpu.async_copy(tok_hbm.at[ids_v.at[1]], rows_v.at[1], sems[1])

    def row(b, carry):
        for j in range(2):
            hc = 2 * b + j
            pltpu.make_async_copy(tok_hbm.at[ids_v.at[hc]], rows_v.at[j], sems[j]).wait()
            for cg in range(3):
                def inner(r, carry_vecs, _j=j, _cg=cg):
                    out = list(carry_vecs)
                    mask_hi = jnp.full((16,), -65536, jnp.int32)
                    sh16 = jnp.full((16,), 16, jnp.int32)
                    for gi in range(8):
                        xi = rows_v[_j, r, pl.ds((_cg * 8 + gi) * 16, 16)]
                        a = lax.bitcast_convert_type(lax.shift_left(xi, sh16), jnp.float32)
                        bb = lax.bitcast_convert_type(lax.bitwise_and(xi, mask_hi), jnp.float32)
                        out[2 * gi] = out[2 * gi] + a
                        out[2 * gi + 1] = out[2 * gi + 1] + bb
                    return tuple(out)
                if j == 0:
                    init = tuple(jnp.zeros((16,), jnp.float32) for _ in range(16))
                else:
                    init = tuple(acc_v[0, pl.ds((cg * 8 + gi) * 32 + half * 16, 16)]
                                 for gi in range(8) for half in range(2))
                res = lax.fori_loop(0, HALF, inner, init)
                for gi in range(8):
                    for half in range(2):
                        acc_v[0, pl.ds((cg * 8 + gi) * 32 + half * 16, 16)] = res[2 * gi + half]
            @pl.when(hc + 2 < 2 * ROWS_PER_W)
            def _():
                pltpu.async_copy(tok_hbm.at[ids_v.at[hc + 2]], rows_v.at[j], sems[j])
        pltpu.sync_copy(acc_v, psum_hbm.at[wid * ROWS_PER_W + b])
        return carry

    lax.fori_loop(0, ROWS_PER_W, row, 0)


def _unpack_tc(w):
    lo = lax.bitcast_convert_type(w << 16, jnp.float32)
    hi = lax.bitcast_convert_type(w & jnp.int32(-65536), jnp.float32)
    return jnp.concatenate([lo, hi], axis=-1)


def _att_body(xh_ref, xt_ref, w1a_ref, w1b_ref, b1_ref, w2_ref, b2_ref, exp4_ref, out_ref):
    xh = _unpack_tc(xh_ref[...])
    xt = _unpack_tc(xt_ref[...])
    hid = jnp.dot(xh, w1a_ref[...], preferred_element_type=jnp.float32)
    hid = hid + jnp.dot(xt, w1b_ref[...], preferred_element_type=jnp.float32)
    hid = jnp.maximum(hid + b1_ref[...], 0.0)
    logits = jnp.dot(hid, w2_ref[...], preferred_element_type=jnp.float32) + b2_ref[...]
    m = jnp.max(logits)
    el = jnp.exp(logits - m)
    el3 = el.reshape(64, K, NF)
    denom = jnp.sum(el3, axis=1, keepdims=True)
    w = (el3 / denom).reshape(2048, NF)
    wt = jnp.dot(w, exp4_ref[...], preferred_element_type=jnp.float32)
    contrib = wt * xt
    out_ref[...] = jnp.sum(contrib.reshape(64, K, 128), axis=1)[None]


def _pack_body(x_ref, out_ref):
    x = x_ref[...]
    lo = lax.bitcast_convert_type(x[:, :H // 2], jnp.uint32)
    hi = lax.bitcast_convert_type(x[:, H // 2:], jnp.uint32)
    half = jnp.uint32(0x8000)
    lo = (lo + half) >> jnp.uint32(16)
    hi = (hi + half) & jnp.uint32(0xFFFF0000)
    out_ref[...] = lax.bitcast_convert_type(lo | hi, jnp.int32)


def _head_body(ps_ref, cnt_ref, poolw_ref, poolb_ref, linw_ref, linb_ref, out_ref):
    pooled = ps_ref[...] / cnt_ref[...]
    p2 = jnp.tanh(jnp.dot(pooled, poolw_ref[...], preferred_element_type=jnp.float32)
                  + poolb_ref[...])
    logits = jnp.dot(p2, linw_ref[...], preferred_element_type=jnp.float32) + linb_ref[...]
    out_ref[...] = jax.nn.sigmoid(logits)


def kernel(users, movies, user_neighbors, movie_neighbors, input_ids, attention_mask,
           node_emb, relation_emb, attW1, attb1, attW2, attb2, tok_emb, poolW, poolb,
           linW, linb):
    f32 = jnp.float32
    bf16 = jnp.bfloat16
    node_flat = node_emb.reshape(N_NODE, NF * DIM)

    segs = [users.astype(jnp.int32), movies.astype(jnp.int32)]
    for nb in (user_neighbors, movie_neighbors):
        for i in range(NL):
            segs.append(nb[:, 0, i, :].reshape(-1).astype(jnp.int32))
            segs.append(nb[:, 2, i, :].reshape(-1).astype(jnp.int32))
    all_idx = jnp.concatenate(segs)
    all_idx = jnp.pad(all_idx, (0, N_PAD - N_REAL)).reshape(NW, CH_PER_W, CHUNK)

    mesh = plsc.VectorSubcoreMesh(core_axis_name="c", subcore_axis_name="s",
                                  num_cores=NC, num_subcores=NS)

    gout_i = pl.kernel(
        _node_gather_body,
        out_type=jax.ShapeDtypeStruct((N_PAD, NF * DIM // 2), jnp.int32),
        mesh=mesh,
        scratch_types=[
            pltpu.VMEM((CH_PER_W, CHUNK), jnp.int32),
            pltpu.VMEM((2, CHUNK, NF * DIM), f32),
            pltpu.VMEM((2, CHUNK, NF * DIM // 2), jnp.int32),
            pltpu.SemaphoreType.DMA,
            pltpu.SemaphoreType.DMA,
        ],
    )(node_flat, all_idx)

    ids2 = input_ids.reshape(NW, 2 * ROWS_PER_W, HALF).astype(jnp.int32)
    nvocab = tok_emb.shape[0]
    tok_i = pl.pallas_call(
        _pack_body,
        grid=(pl.cdiv(nvocab, 1024),),
        in_specs=[pl.BlockSpec((1024, H), lambda i: (i, 0))],
        out_specs=pl.BlockSpec((1024, H // 2), lambda i: (i, 0)),
        out_shape=jax.ShapeDtypeStruct((nvocab, H // 2), jnp.int32),
    )(tok_emb)
    psum = pl.kernel(
        _lm_pool_body,
        out_type=jax.ShapeDtypeStruct((B, 1, H), f32),
        mesh=mesh,
        scratch_types=[
            pltpu.VMEM((2 * ROWS_PER_W, HALF), jnp.int32),
            pltpu.VMEM((2, HALF, H // 2), jnp.int32),
            pltpu.VMEM((1, H), f32),
            pltpu.SemaphoreType.DMA,
            pltpu.SemaphoreType.DMA,
        ],
    )(tok_i, ids2).reshape(B, H)

    w1a = jnp.kron(jnp.eye(NF, dtype=f32), attW1[:DIM, :])
    w1b = jnp.kron(jnp.eye(NF, dtype=f32), attW1[DIM:, :])
    b1t = jnp.tile(attb1, (NF,))[None, :]
    w2t = jnp.kron(jnp.eye(NF, dtype=f32), attW2)
    b2t = jnp.tile(attb2, (NF,))[None, :]
    exp4 = jnp.kron(jnp.eye(NF, dtype=f32), jnp.ones((1, DIM), f32))

    att = pl.pallas_call(
        _att_body,
        grid=(4, 16),
        in_specs=[
            pl.BlockSpec((2048, 64), lambda lb, j: (1 + 32 * lb + j, 0)),
            pl.BlockSpec((2048, 64), lambda lb, j: (17 + 32 * lb + j, 0)),
            pl.BlockSpec((128, 128), lambda lb, j: (0, 0)),
            pl.BlockSpec((128, 128), lambda lb, j: (0, 0)),
            pl.BlockSpec((1, 128), lambda lb, j: (0, 0)),
            pl.BlockSpec((128, 4), lambda lb, j: (0, 0)),
            pl.BlockSpec((1, 4), lambda lb, j: (0, 0)),
            pl.BlockSpec((4, 128), lambda lb, j: (0, 0)),
        ],
        out_specs=pl.BlockSpec((1, 64, 128), lambda lb, j: (lb, j, 0)),
        out_shape=jax.ShapeDtypeStruct((4, B, 128), f32),
    )(gout_i, gout_i, w1a, w1b, b1t, w2t, b2t, exp4)

    counts = jnp.maximum(jnp.sum(attention_mask, axis=1, keepdims=True), 1).astype(f32)
    linw_pad = jnp.zeros((H, 128), f32).at[:, :2].set(linW)
    linb_pad = jnp.zeros((1, 128), f32).at[0, :2].set(linb)
    g = jnp.arange(H) // 32
    r = jnp.arange(H) % 32
    perm = jnp.where(r < 16, 16 * g + r, H // 2 + 16 * g + (r - 16))
    poolw_perm = poolW[perm, :]
    proba_pad = pl.pallas_call(
        _head_body,
        out_shape=jax.ShapeDtypeStruct((B, 128), f32),
    )(psum, counts, poolw_perm, poolb[None, :], linw_pad, linb_pad)
    proba = proba_pad[:, :2]

    self_uv = _unpack_tc(gout_i[0:2 * B])
    e_u = jnp.stack([self_uv[0:B], att[0], att[1]], axis=1).reshape(B, NL + 1, NF, DIM)
    e_v = jnp.stack([self_uv[B:2 * B], att[2], att[3]], axis=1).reshape(B, NL + 1, NF, DIM)
    return (proba, e_u, e_v)


# speedup vs baseline: 1.9005x; 1.9005x over previous
import functools

import jax
import jax.numpy as jnp
from jax import lax
from jax.experimental import pallas as pl
from jax.experimental.pallas import tpu as pltpu
from jax.experimental.pallas import tpu_sc as plsc

N_NODE = 100000
DIM = 32
NF = 4
NL = 2
B = 1024
K = 32
SEQ = 128
H = 768

NC = 2
NS = 16
NW = NC * NS

N_REAL = 2 * B + 8 * B * K
CHUNK = 128
CH_PER_W = 66
N_PAD = NW * CH_PER_W * CHUNK

ROWS_PER_W = B // NW
HALF = SEQ // 2


def _node_gather_body(node_hbm, idx_hbm, gout_hbm, idx_v, rows_v, out_v, sem0, sem1):
    wid = lax.axis_index("s") * NC + lax.axis_index("c")
    sems = (sem0, sem1)
    pltpu.sync_copy(idx_hbm.at[wid], idx_v)
    pltpu.async_copy(node_hbm.at[idx_v.at[0]], rows_v.at[0], sems[0])
    rnd = jnp.full((16,), 32768, jnp.int32)
    mask_hi = jnp.full((16,), -65536, jnp.int32)
    sh16 = jnp.full((16,), 16, jnp.int32)

    def pair(g, carry):
        for b in range(2):
            c = 2 * g + b
            pltpu.make_async_copy(node_hbm.at[idx_v.at[c]], rows_v.at[b], sems[b]).wait()

            @pl.when(c + 1 < CH_PER_W)
            def _():
                pltpu.async_copy(node_hbm.at[idx_v.at[c + 1]], rows_v.at[1 - b],
                                 sems[1 - b])

            def packrow(r, cc, _b=b):
                for q in range(4):
                    lo = lax.bitcast_convert_type(
                        rows_v[_b, r, pl.ds(q * 16, 16)], jnp.int32)
                    hi = lax.bitcast_convert_type(
                        rows_v[_b, r, pl.ds(64 + q * 16, 16)], jnp.int32)
                    lo = lax.shift_right_logical(lo + rnd, sh16)
                    hi = lax.bitwise_and(hi + rnd, mask_hi)
                    out_v[_b, r, pl.ds(q * 16, 16)] = lax.bitwise_or(lo, hi)
                return cc
            lax.fori_loop(0, CHUNK, packrow, 0)

            base = wid * (CH_PER_W * CHUNK) + c * CHUNK
            pltpu.sync_copy(out_v.at[b], gout_hbm.at[pl.ds(base, CHUNK)])
        return carry

    lax.fori_loop(0, CH_PER_W // 2, pair, 0)


def _lm_pool_body(tok_hbm, ids_hbm, psum_hbm, ids_v, rows_v, acc_v, sem0, sem1):
    wid = lax.axis_index("s") * NC + lax.axis_index("c")
    sems = (sem0, sem1)
    pltpu.sync_copy(ids_hbm.at[wid], ids_v)

    pltpu.async_copy(tok_hbm.at[ids_v.at[0]], rows_v.at[0], sems[0])
    pltpu.async_copy(tok_hbm.at[ids_v.at[1]], rows_v.at[1], sems[1])

    def row(b, carry):
        for j in range(2):
            hc = 2 * b + j
            pltpu.make_async_copy(tok_hbm.at[ids_v.at[hc]], rows_v.at[j], sems[j]).wait()
            for cg in range(3):
                def inner(r, carry_vecs, _j=j, _cg=cg):
                    out = list(carry_vecs)
                    mask_hi = jnp.full((16,), -65536, jnp.int32)
                    sh16 = jnp.full((16,), 16, jnp.int32)
                    for gi in range(8):
                        xi = rows_v[_j, r, pl.ds((_cg * 8 + gi) * 16, 16)]
                        a = lax.bitcast_convert_type(lax.shift_left(xi, sh16), jnp.float32)
                        bb = lax.bitcast_convert_type(lax.bitwise_and(xi, mask_hi), jnp.float32)
                        out[2 * gi] = out[2 * gi] + a
                        out[2 * gi + 1] = out[2 * gi + 1] + bb
                    return tuple(out)
                if j == 0:
                    init = tuple(jnp.zeros((16,), jnp.float32) for _ in range(16))
                else:
                    init = tuple(acc_v[0, pl.ds((cg * 8 + gi) * 32 + half * 16, 16)]
                                 for gi in range(8) for half in range(2))
                res = lax.fori_loop(0, HALF, inner, init)
                for gi in range(8):
                    for half in range(2):
                        acc_v[0, pl.ds((cg * 8 + gi) * 32 + half * 16, 16)] = res[2 * gi + half]
            @pl.when(hc + 2 < 2 * ROWS_PER_W)
            def _():
                pltpu.async_copy(tok_hbm.at[ids_v.at[hc + 2]], rows_v.at[j], sems[j])
        pltpu.sync_copy(acc_v, psum_hbm.at[wid * ROWS_PER_W + b])
        return carry

    lax.fori_loop(0, ROWS_PER_W, row, 0)


def _unpack_tc(w):
    lo = lax.bitcast_convert_type(w << 16, jnp.float32)
    hi = lax.bitcast_convert_type(w & jnp.int32(-65536), jnp.float32)
    return jnp.concatenate([lo, hi], axis=-1)


def _att_body(xh_ref, xt_ref, w1a_ref, w1b_ref, b1_ref, w2_ref, b2_ref, exp4_ref, out_ref):
    xh = _unpack_tc(xh_ref[...])
    xt = _unpack_tc(xt_ref[...])
    hid = jnp.dot(xh, w1a_ref[...], preferred_element_type=jnp.float32)
    hid = hid + jnp.dot(xt, w1b_ref[...], preferred_element_type=jnp.float32)
    hid = jnp.maximum(hid + b1_ref[...], 0.0)
    logits = jnp.dot(hid, w2_ref[...], preferred_element_type=jnp.float32) + b2_ref[...]
    m = jnp.max(logits)
    el = jnp.exp(logits - m)
    el3 = el.reshape(64, K, NF)
    denom = jnp.sum(el3, axis=1, keepdims=True)
    w = (el3 / denom).reshape(2048, NF)
    wt = jnp.dot(w, exp4_ref[...], preferred_element_type=jnp.float32)
    contrib = wt * xt
    out_ref[...] = jnp.sum(contrib.reshape(64, K, 128), axis=1)[None]


def _pack_body(x_ref, out_ref):
    x = x_ref[...]
    lo = lax.bitcast_convert_type(x[:, :H // 2], jnp.uint32)
    hi = lax.bitcast_convert_type(x[:, H // 2:], jnp.uint32)
    half = jnp.uint32(0x8000)
    lo = (lo + half) >> jnp.uint32(16)
    hi = (hi + half) & jnp.uint32(0xFFFF0000)
    out_ref[...] = lax.bitcast_convert_type(lo | hi, jnp.int32)


def _head_body(ps_ref, cnt_ref, poolw_ref, poolb_ref, linw_ref, linb_ref, out_ref):
    pooled = ps_ref[...] / cnt_ref[...]
    p2 = jnp.tanh(jnp.dot(pooled, poolw_ref[...], preferred_element_type=jnp.float32)
                  + poolb_ref[...])
    logits = jnp.dot(p2, linw_ref[...], preferred_element_type=jnp.float32) + linb_ref[...]
    out_ref[...] = jax.nn.sigmoid(logits)


def kernel(users, movies, user_neighbors, movie_neighbors, input_ids, attention_mask,
           node_emb, relation_emb, attW1, attb1, attW2, attb2, tok_emb, poolW, poolb,
           linW, linb):
    f32 = jnp.float32
    bf16 = jnp.bfloat16
    node_flat = node_emb.reshape(N_NODE, NF * DIM)

    segs = [users.astype(jnp.int32), movies.astype(jnp.int32)]
    for nb in (user_neighbors, movie_neighbors):
        for i in range(NL):
            segs.append(nb[:, 0, i, :].reshape(-1).astype(jnp.int32))
            segs.append(nb[:, 2, i, :].reshape(-1).astype(jnp.int32))
    pad_idx = (jnp.arange(N_PAD - N_REAL, dtype=jnp.int32) * 97) % N_NODE
    all_idx = jnp.concatenate(segs + [pad_idx]).reshape(NW, CH_PER_W, CHUNK)

    mesh = plsc.VectorSubcoreMesh(core_axis_name="c", subcore_axis_name="s",
                                  num_cores=NC, num_subcores=NS)

    gout_i = pl.kernel(
        _node_gather_body,
        out_type=jax.ShapeDtypeStruct((N_PAD, NF * DIM // 2), jnp.int32),
        mesh=mesh,
        scratch_types=[
            pltpu.VMEM((CH_PER_W, CHUNK), jnp.int32),
            pltpu.VMEM((2, CHUNK, NF * DIM), f32),
            pltpu.VMEM((2, CHUNK, NF * DIM // 2), jnp.int32),
            pltpu.SemaphoreType.DMA,
            pltpu.SemaphoreType.DMA,
        ],
    )(node_flat, all_idx)

    ids2 = input_ids.reshape(NW, 2 * ROWS_PER_W, HALF).astype(jnp.int32)
    nvocab = tok_emb.shape[0]
    tok_i = pl.pallas_call(
        _pack_body,
        grid=(pl.cdiv(nvocab, 1024),),
        in_specs=[pl.BlockSpec((1024, H), lambda i: (i, 0))],
        out_specs=pl.BlockSpec((1024, H // 2), lambda i: (i, 0)),
        out_shape=jax.ShapeDtypeStruct((nvocab, H // 2), jnp.int32),
    )(tok_emb)
    psum = pl.kernel(
        _lm_pool_body,
        out_type=jax.ShapeDtypeStruct((B, 1, H), f32),
        mesh=mesh,
        scratch_types=[
            pltpu.VMEM((2 * ROWS_PER_W, HALF), jnp.int32),
            pltpu.VMEM((2, HALF, H // 2), jnp.int32),
            pltpu.VMEM((1, H), f32),
            pltpu.SemaphoreType.DMA,
            pltpu.SemaphoreType.DMA,
        ],
    )(tok_i, ids2).reshape(B, H)

    w1a = jnp.kron(jnp.eye(NF, dtype=f32), attW1[:DIM, :])
    w1b = jnp.kron(jnp.eye(NF, dtype=f32), attW1[DIM:, :])
    b1t = jnp.tile(attb1, (NF,))[None, :]
    w2t = jnp.kron(jnp.eye(NF, dtype=f32), attW2)
    b2t = jnp.tile(attb2, (NF,))[None, :]
    exp4 = jnp.kron(jnp.eye(NF, dtype=f32), jnp.ones((1, DIM), f32))

    att = pl.pallas_call(
        _att_body,
        grid=(4, 16),
        in_specs=[
            pl.BlockSpec((2048, 64), lambda lb, j: (1 + 32 * lb + j, 0)),
            pl.BlockSpec((2048, 64), lambda lb, j: (17 + 32 * lb + j, 0)),
            pl.BlockSpec((128, 128), lambda lb, j: (0, 0)),
            pl.BlockSpec((128, 128), lambda lb, j: (0, 0)),
            pl.BlockSpec((1, 128), lambda lb, j: (0, 0)),
            pl.BlockSpec((128, 4), lambda lb, j: (0, 0)),
            pl.BlockSpec((1, 4), lambda lb, j: (0, 0)),
            pl.BlockSpec((4, 128), lambda lb, j: (0, 0)),
        ],
        out_specs=pl.BlockSpec((1, 64, 128), lambda lb, j: (lb, j, 0)),
        out_shape=jax.ShapeDtypeStruct((4, B, 128), f32),
    )(gout_i, gout_i, w1a, w1b, b1t, w2t, b2t, exp4)

    counts = jnp.maximum(jnp.sum(attention_mask, axis=1, keepdims=True), 1).astype(f32)
    linw_pad = jnp.zeros((H, 128), f32).at[:, :2].set(linW)
    linb_pad = jnp.zeros((1, 128), f32).at[0, :2].set(linb)
    g = jnp.arange(H) // 32
    r = jnp.arange(H) % 32
    perm = jnp.where(r < 16, 16 * g + r, H // 2 + 16 * g + (r - 16))
    poolw_perm = poolW[perm, :]
    proba_pad = pl.pallas_call(
        _head_body,
        out_shape=jax.ShapeDtypeStruct((B, 128), f32),
    )(psum, counts, poolw_perm, poolb[None, :], linw_pad, linb_pad)
    proba = proba_pad[:, :2]

    self_uv = _unpack_tc(gout_i[0:2 * B])
    e_u = jnp.stack([self_uv[0:B], att[0], att[1]], axis=1).reshape(B, NL + 1, NF, DIM)
    e_v = jnp.stack([self_uv[B:2 * B], att[2], att[3]], axis=1).reshape(B, NL + 1, NF, DIM)
    return (proba, e_u, e_v)


# speedup vs baseline: 1.9518x; 1.0270x over previous
import functools

import jax
import jax.numpy as jnp
from jax import lax
from jax.experimental import pallas as pl
from jax.experimental.pallas import tpu as pltpu
from jax.experimental.pallas import tpu_sc as plsc

N_NODE = 100000
DIM = 32
NF = 4
NL = 2
B = 1024
K = 32
SEQ = 128
H = 768

NC = 2
NS = 16
NW = NC * NS

N_REAL = 2 * B + 8 * B * K
CHUNK = 128
CH_PER_W = 66
N_PAD = NW * CH_PER_W * CHUNK

ROWS_PER_W = B // NW
HALF = SEQ // 2


NBUF = 3


def _node_gather_body(node_hbm, idx_hbm, gout_hbm, idx_v, rows_v, out_v, *sems):
    wid = lax.axis_index("s") * NC + lax.axis_index("c")
    pltpu.sync_copy(idx_hbm.at[wid], idx_v)
    for b in range(NBUF - 1):
        pltpu.async_copy(node_hbm.at[idx_v.at[b]], rows_v.at[b], sems[b])
    rnd = jnp.full((16,), 32768, jnp.int32)
    mask_hi = jnp.full((16,), -65536, jnp.int32)
    sh16 = jnp.full((16,), 16, jnp.int32)

    def group(g, carry):
        for b in range(NBUF):
            c = NBUF * g + b

            @pl.when(c + NBUF - 1 < CH_PER_W)
            def _():
                pltpu.async_copy(node_hbm.at[idx_v.at[c + NBUF - 1]],
                                 rows_v.at[(b + NBUF - 1) % NBUF],
                                 sems[(b + NBUF - 1) % NBUF])
            pltpu.make_async_copy(node_hbm.at[idx_v.at[c]], rows_v.at[b], sems[b]).wait()

            def packrow(r, cc, _b=b):
                for q in range(4):
                    lo = lax.bitcast_convert_type(
                        rows_v[_b, r, pl.ds(q * 16, 16)], jnp.int32)
                    hi = lax.bitcast_convert_type(
                        rows_v[_b, r, pl.ds(64 + q * 16, 16)], jnp.int32)
                    lo = lax.shift_right_logical(lo + rnd, sh16)
                    hi = lax.bitwise_and(hi + rnd, mask_hi)
                    out_v[_b, r, pl.ds(q * 16, 16)] = lax.bitwise_or(lo, hi)
                return cc
            lax.fori_loop(0, CHUNK, packrow, 0)

            base = wid * (CH_PER_W * CHUNK) + c * CHUNK
            pltpu.sync_copy(out_v.at[b], gout_hbm.at[pl.ds(base, CHUNK)])
        return carry

    lax.fori_loop(0, CH_PER_W // NBUF, group, 0)


def _lm_pool_body(tok_hbm, ids_hbm, psum_hbm, ids_v, rows_v, acc_v, sem0, sem1):
    wid = lax.axis_index("s") * NC + lax.axis_index("c")
    sems = (sem0, sem1)
    pltpu.sync_copy(ids_hbm.at[wid], ids_v)

    pltpu.async_copy(tok_hbm.at[ids_v.at[0]], rows_v.at[0], sems[0])
    pltpu.async_copy(tok_hbm.at[ids_v.at[1]], rows_v.at[1], sems[1])

    def row(b, carry):
        for j in range(2):
            hc = 2 * b + j
            pltpu.make_async_copy(tok_hbm.at[ids_v.at[hc]], rows_v.at[j], sems[j]).wait()
            for cg in range(3):
                def inner(r, carry_vecs, _j=j, _cg=cg):
                    out = list(carry_vecs)
                    mask_hi = jnp.full((16,), -65536, jnp.int32)
                    sh16 = jnp.full((16,), 16, jnp.int32)
                    for gi in range(8):
                        xi = rows_v[_j, r, pl.ds((_cg * 8 + gi) * 16, 16)]
                        a = lax.bitcast_convert_type(lax.shift_left(xi, sh16), jnp.float32)
                        bb = lax.bitcast_convert_type(lax.bitwise_and(xi, mask_hi), jnp.float32)
                        out[2 * gi] = out[2 * gi] + a
                        out[2 * gi + 1] = out[2 * gi + 1] + bb
                    return tuple(out)
                if j == 0:
                    init = tuple(jnp.zeros((16,), jnp.float32) for _ in range(16))
                else:
                    init = tuple(acc_v[0, pl.ds((cg * 8 + gi) * 32 + half * 16, 16)]
                                 for gi in range(8) for half in range(2))
                res = lax.fori_loop(0, HALF, inner, init)
                for gi in range(8):
                    for half in range(2):
                        acc_v[0, pl.ds((cg * 8 + gi) * 32 + half * 16, 16)] = res[2 * gi + half]
            @pl.when(hc + 2 < 2 * ROWS_PER_W)
            def _():
                pltpu.async_copy(tok_hbm.at[ids_v.at[hc + 2]], rows_v.at[j], sems[j])
        pltpu.sync_copy(acc_v, psum_hbm.at[wid * ROWS_PER_W + b])
        return carry

    lax.fori_loop(0, ROWS_PER_W, row, 0)


def _unpack_tc(w):
    lo = lax.bitcast_convert_type(w << 16, jnp.float32)
    hi = lax.bitcast_convert_type(w & jnp.int32(-65536), jnp.float32)
    return jnp.concatenate([lo, hi], axis=-1)


def _att_body(xh_ref, xt_ref, w1a_ref, w1b_ref, b1_ref, w2_ref, b2_ref, exp4_ref, out_ref):
    xh = _unpack_tc(xh_ref[...])
    xt = _unpack_tc(xt_ref[...])
    hid = jnp.dot(xh, w1a_ref[...], preferred_element_type=jnp.float32)
    hid = hid + jnp.dot(xt, w1b_ref[...], preferred_element_type=jnp.float32)
    hid = jnp.maximum(hid + b1_ref[...], 0.0)
    logits = jnp.dot(hid, w2_ref[...], preferred_element_type=jnp.float32) + b2_ref[...]
    m = jnp.max(logits)
    el = jnp.exp(logits - m)
    el3 = el.reshape(64, K, NF)
    denom = jnp.sum(el3, axis=1, keepdims=True)
    w = (el3 / denom).reshape(2048, NF)
    wt = jnp.dot(w, exp4_ref[...], preferred_element_type=jnp.float32)
    contrib = wt * xt
    out_ref[...] = jnp.sum(contrib.reshape(64, K, 128), axis=1)[None]


def _pack_body(x_ref, out_ref):
    x = x_ref[...]
    lo = lax.bitcast_convert_type(x[:, :H // 2], jnp.uint32)
    hi = lax.bitcast_convert_type(x[:, H // 2:], jnp.uint32)
    half = jnp.uint32(0x8000)
    lo = (lo + half) >> jnp.uint32(16)
    hi = (hi + half) & jnp.uint32(0xFFFF0000)
    out_ref[...] = lax.bitcast_convert_type(lo | hi, jnp.int32)


def _head_body(ps_ref, cnt_ref, poolw_ref, poolb_ref, linw_ref, linb_ref, out_ref):
    pooled = ps_ref[...] / cnt_ref[...]
    p2 = jnp.tanh(jnp.dot(pooled, poolw_ref[...], preferred_element_type=jnp.float32)
                  + poolb_ref[...])
    logits = jnp.dot(p2, linw_ref[...], preferred_element_type=jnp.float32) + linb_ref[...]
    out_ref[...] = jax.nn.sigmoid(logits)


def kernel(users, movies, user_neighbors, movie_neighbors, input_ids, attention_mask,
           node_emb, relation_emb, attW1, attb1, attW2, attb2, tok_emb, poolW, poolb,
           linW, linb):
    f32 = jnp.float32
    bf16 = jnp.bfloat16
    node_flat = node_emb.reshape(N_NODE, NF * DIM)

    segs = [users.astype(jnp.int32), movies.astype(jnp.int32)]
    for nb in (user_neighbors, movie_neighbors):
        for i in range(NL):
            segs.append(nb[:, 0, i, :].reshape(-1).astype(jnp.int32))
            segs.append(nb[:, 2, i, :].reshape(-1).astype(jnp.int32))
    pad_idx = (jnp.arange(N_PAD - N_REAL, dtype=jnp.int32) * 97) % N_NODE
    all_idx = jnp.concatenate(segs + [pad_idx]).reshape(NW, CH_PER_W, CHUNK)

    mesh = plsc.VectorSubcoreMesh(core_axis_name="c", subcore_axis_name="s",
                                  num_cores=NC, num_subcores=NS)

    gout_i = pl.kernel(
        _node_gather_body,
        out_type=jax.ShapeDtypeStruct((N_PAD, NF * DIM // 2), jnp.int32),
        mesh=mesh,
        scratch_types=[
            pltpu.VMEM((CH_PER_W, CHUNK), jnp.int32),
            pltpu.VMEM((NBUF, CHUNK, NF * DIM), f32),
            pltpu.VMEM((NBUF, CHUNK, NF * DIM // 2), jnp.int32),
        ] + [pltpu.SemaphoreType.DMA] * NBUF,
    )(node_flat, all_idx)

    ids2 = input_ids.reshape(NW, 2 * ROWS_PER_W, HALF).astype(jnp.int32)
    nvocab = tok_emb.shape[0]
    tok_i = pl.pallas_call(
        _pack_body,
        grid=(pl.cdiv(nvocab, 1024),),
        in_specs=[pl.BlockSpec((1024, H), lambda i: (i, 0))],
        out_specs=pl.BlockSpec((1024, H // 2), lambda i: (i, 0)),
        out_shape=jax.ShapeDtypeStruct((nvocab, H // 2), jnp.int32),
    )(tok_emb)
    psum = pl.kernel(
        _lm_pool_body,
        out_type=jax.ShapeDtypeStruct((B, 1, H), f32),
        mesh=mesh,
        scratch_types=[
            pltpu.VMEM((2 * ROWS_PER_W, HALF), jnp.int32),
            pltpu.VMEM((2, HALF, H // 2), jnp.int32),
            pltpu.VMEM((1, H), f32),
            pltpu.SemaphoreType.DMA,
            pltpu.SemaphoreType.DMA,
        ],
    )(tok_i, ids2).reshape(B, H)

    w1a = jnp.kron(jnp.eye(NF, dtype=f32), attW1[:DIM, :])
    w1b = jnp.kron(jnp.eye(NF, dtype=f32), attW1[DIM:, :])
    b1t = jnp.tile(attb1, (NF,))[None, :]
    w2t = jnp.kron(jnp.eye(NF, dtype=f32), attW2)
    b2t = jnp.tile(attb2, (NF,))[None, :]
    exp4 = jnp.kron(jnp.eye(NF, dtype=f32), jnp.ones((1, DIM), f32))

    att = pl.pallas_call(
        _att_body,
        grid=(4, 16),
        in_specs=[
            pl.BlockSpec((2048, 64), lambda lb, j: (1 + 32 * lb + j, 0)),
            pl.BlockSpec((2048, 64), lambda lb, j: (17 + 32 * lb + j, 0)),
            pl.BlockSpec((128, 128), lambda lb, j: (0, 0)),
            pl.BlockSpec((128, 128), lambda lb, j: (0, 0)),
            pl.BlockSpec((1, 128), lambda lb, j: (0, 0)),
            pl.BlockSpec((128, 4), lambda lb, j: (0, 0)),
            pl.BlockSpec((1, 4), lambda lb, j: (0, 0)),
            pl.BlockSpec((4, 128), lambda lb, j: (0, 0)),
        ],
        out_specs=pl.BlockSpec((1, 64, 128), lambda lb, j: (lb, j, 0)),
        out_shape=jax.ShapeDtypeStruct((4, B, 128), f32),
    )(gout_i, gout_i, w1a, w1b, b1t, w2t, b2t, exp4)

    counts = jnp.maximum(jnp.sum(attention_mask, axis=1, keepdims=True), 1).astype(f32)
    linw_pad = jnp.zeros((H, 128), f32).at[:, :2].set(linW)
    linb_pad = jnp.zeros((1, 128), f32).at[0, :2].set(linb)
    g = jnp.arange(H) // 32
    r = jnp.arange(H) % 32
    perm = jnp.where(r < 16, 16 * g + r, H // 2 + 16 * g + (r - 16))
    poolw_perm = poolW[perm, :]
    proba_pad = pl.pallas_call(
        _head_body,
        out_shape=jax.ShapeDtypeStruct((B, 128), f32),
    )(psum, counts, poolw_perm, poolb[None, :], linw_pad, linb_pad)
    proba = proba_pad[:, :2]

    self_uv = _unpack_tc(gout_i[0:2 * B])
    e_u = jnp.stack([self_uv[0:B], att[0], att[1]], axis=1).reshape(B, NL + 1, NF, DIM)
    e_v = jnp.stack([self_uv[B:2 * B], att[2], att[3]], axis=1).reshape(B, NL + 1, NF, DIM)
    return (proba, e_u, e_v)


# speedup vs baseline: 1.9606x; 1.0045x over previous
import functools

import jax
import jax.numpy as jnp
from jax import lax
from jax.experimental import pallas as pl
from jax.experimental.pallas import tpu as pltpu
from jax.experimental.pallas import tpu_sc as plsc

N_NODE = 100000
DIM = 32
NF = 4
NL = 2
B = 1024
K = 32
SEQ = 128
H = 768

NC = 2
NS = 16
NW = NC * NS

N_REAL = 2 * B + 8 * B * K
CHUNK = 128
CH_PER_W = 66
N_PAD = NW * CH_PER_W * CHUNK

ROWS_PER_W = B // NW
HALF = SEQ // 2


NBUF = 3


def _node_gather_body(node_hbm, idx_hbm, gout_hbm, idx_v, rows_v, out_v, *sems):
    rsems, wsems = sems[:NBUF], sems[NBUF:]
    wid = lax.axis_index("s") * NC + lax.axis_index("c")
    pltpu.sync_copy(idx_hbm.at[wid], idx_v)
    for b in range(NBUF - 1):
        pltpu.async_copy(node_hbm.at[idx_v.at[b]], rows_v.at[b], rsems[b])
    rnd = jnp.full((16,), 32768, jnp.int32)
    mask_hi = jnp.full((16,), -65536, jnp.int32)
    sh16 = jnp.full((16,), 16, jnp.int32)
    nwc = NBUF * 2

    def group(g, carry):
        for b2 in range(nwc):
            c = nwc * g + b2
            b = b2 % NBUF
            wb = b2 % 2

            @pl.when(c + NBUF - 1 < CH_PER_W)
            def _():
                pltpu.async_copy(node_hbm.at[idx_v.at[c + NBUF - 1]],
                                 rows_v.at[(b + NBUF - 1) % NBUF],
                                 rsems[(b + NBUF - 1) % NBUF])
            pltpu.make_async_copy(node_hbm.at[idx_v.at[c]], rows_v.at[b], rsems[b]).wait()

            @pl.when(c >= 2)
            def _():
                pltpu.make_async_copy(
                    out_v.at[wb], gout_hbm.at[pl.ds(0, CHUNK)], wsems[wb]).wait()

            def packrow(r, cc, _b=b, _wb=wb):
                for q in range(4):
                    lo = lax.bitcast_convert_type(
                        rows_v[_b, r, pl.ds(q * 16, 16)], jnp.int32)
                    hi = lax.bitcast_convert_type(
                        rows_v[_b, r, pl.ds(64 + q * 16, 16)], jnp.int32)
                    lo = lax.shift_right_logical(lo + rnd, sh16)
                    hi = lax.bitwise_and(hi + rnd, mask_hi)
                    out_v[_wb, r, pl.ds(q * 16, 16)] = lax.bitwise_or(lo, hi)
                return cc
            lax.fori_loop(0, CHUNK, packrow, 0)

            base = wid * (CH_PER_W * CHUNK) + c * CHUNK
            pltpu.async_copy(out_v.at[wb], gout_hbm.at[pl.ds(base, CHUNK)], wsems[wb])
        return carry

    lax.fori_loop(0, CH_PER_W // nwc, group, 0)
    for wb in range(2):
        pltpu.make_async_copy(out_v.at[wb], gout_hbm.at[pl.ds(0, CHUNK)], wsems[wb]).wait()


def _lm_pool_body(tok_hbm, ids_hbm, psum_hbm, ids_v, rows_v, acc_v, sem0, sem1):
    wid = lax.axis_index("s") * NC + lax.axis_index("c")
    sems = (sem0, sem1)
    pltpu.sync_copy(ids_hbm.at[wid], ids_v)

    pltpu.async_copy(tok_hbm.at[ids_v.at[0]], rows_v.at[0], sems[0])
    pltpu.async_copy(tok_hbm.at[ids_v.at[1]], rows_v.at[1], sems[1])

    def row(b, carry):
        for j in range(2):
            hc = 2 * b + j
            pltpu.make_async_copy(tok_hbm.at[ids_v.at[hc]], rows_v.at[j], sems[j]).wait()
            for cg in range(3):
                def inner(r, carry_vecs, _j=j, _cg=cg):
                    out = list(carry_vecs)
                    mask_hi = jnp.full((16,), -65536, jnp.int32)
                    sh16 = jnp.full((16,), 16, jnp.int32)
                    for gi in range(8):
                        xi = rows_v[_j, r, pl.ds((_cg * 8 + gi) * 16, 16)]
                        a = lax.bitcast_convert_type(lax.shift_left(xi, sh16), jnp.float32)
                        bb = lax.bitcast_convert_type(lax.bitwise_and(xi, mask_hi), jnp.float32)
                        out[2 * gi] = out[2 * gi] + a
                        out[2 * gi + 1] = out[2 * gi + 1] + bb
                    return tuple(out)
                if j == 0:
                    init = tuple(jnp.zeros((16,), jnp.float32) for _ in range(16))
                else:
                    init = tuple(acc_v[0, pl.ds((cg * 8 + gi) * 32 + half * 16, 16)]
                                 for gi in range(8) for half in range(2))
                res = lax.fori_loop(0, HALF, inner, init)
                for gi in range(8):
                    for half in range(2):
                        acc_v[0, pl.ds((cg * 8 + gi) * 32 + half * 16, 16)] = res[2 * gi + half]
            @pl.when(hc + 2 < 2 * ROWS_PER_W)
            def _():
                pltpu.async_copy(tok_hbm.at[ids_v.at[hc + 2]], rows_v.at[j], sems[j])
        pltpu.sync_copy(acc_v, psum_hbm.at[wid * ROWS_PER_W + b])
        return carry

    lax.fori_loop(0, ROWS_PER_W, row, 0)


def _unpack_tc(w):
    lo = lax.bitcast_convert_type(w << 16, jnp.float32)
    hi = lax.bitcast_convert_type(w & jnp.int32(-65536), jnp.float32)
    return jnp.concatenate([lo, hi], axis=-1)


def _att_body(xh_ref, xt_ref, w1a_ref, w1b_ref, b1_ref, w2_ref, b2_ref, exp4_ref, out_ref):
    xh = _unpack_tc(xh_ref[...])
    xt = _unpack_tc(xt_ref[...])
    hid = jnp.dot(xh, w1a_ref[...], preferred_element_type=jnp.float32)
    hid = hid + jnp.dot(xt, w1b_ref[...], preferred_element_type=jnp.float32)
    hid = jnp.maximum(hid + b1_ref[...], 0.0)
    logits = jnp.dot(hid, w2_ref[...], preferred_element_type=jnp.float32) + b2_ref[...]
    m = jnp.max(logits)
    el = jnp.exp(logits - m)
    el3 = el.reshape(64, K, NF)
    denom = jnp.sum(el3, axis=1, keepdims=True)
    w = (el3 / denom).reshape(2048, NF)
    wt = jnp.dot(w, exp4_ref[...], preferred_element_type=jnp.float32)
    contrib = wt * xt
    out_ref[...] = jnp.sum(contrib.reshape(64, K, 128), axis=1)[None]


def _pack_body(x_ref, out_ref):
    x = x_ref[...]
    lo = lax.bitcast_convert_type(x[:, :H // 2], jnp.uint32)
    hi = lax.bitcast_convert_type(x[:, H // 2:], jnp.uint32)
    half = jnp.uint32(0x8000)
    lo = (lo + half) >> jnp.uint32(16)
    hi = (hi + half) & jnp.uint32(0xFFFF0000)
    out_ref[...] = lax.bitcast_convert_type(lo | hi, jnp.int32)


def _head_body(ps_ref, cnt_ref, poolw_ref, poolb_ref, linw_ref, linb_ref, out_ref):
    pooled = ps_ref[...] / cnt_ref[...]
    p2 = jnp.tanh(jnp.dot(pooled, poolw_ref[...], preferred_element_type=jnp.float32)
                  + poolb_ref[...])
    logits = jnp.dot(p2, linw_ref[...], preferred_element_type=jnp.float32) + linb_ref[...]
    out_ref[...] = jax.nn.sigmoid(logits)


def kernel(users, movies, user_neighbors, movie_neighbors, input_ids, attention_mask,
           node_emb, relation_emb, attW1, attb1, attW2, attb2, tok_emb, poolW, poolb,
           linW, linb):
    f32 = jnp.float32
    bf16 = jnp.bfloat16
    node_flat = node_emb.reshape(N_NODE, NF * DIM)

    segs = [users.astype(jnp.int32), movies.astype(jnp.int32)]
    for nb in (user_neighbors, movie_neighbors):
        for i in range(NL):
            segs.append(nb[:, 0, i, :].reshape(-1).astype(jnp.int32))
            segs.append(nb[:, 2, i, :].reshape(-1).astype(jnp.int32))
    pad_idx = (jnp.arange(N_PAD - N_REAL, dtype=jnp.int32) * 97) % N_NODE
    all_idx = jnp.concatenate(segs + [pad_idx]).reshape(NW, CH_PER_W, CHUNK)

    mesh = plsc.VectorSubcoreMesh(core_axis_name="c", subcore_axis_name="s",
                                  num_cores=NC, num_subcores=NS)

    gout_i = pl.kernel(
        _node_gather_body,
        out_type=jax.ShapeDtypeStruct((N_PAD, NF * DIM // 2), jnp.int32),
        mesh=mesh,
        scratch_types=[
            pltpu.VMEM((CH_PER_W, CHUNK), jnp.int32),
            pltpu.VMEM((NBUF, CHUNK, NF * DIM), f32),
            pltpu.VMEM((2, CHUNK, NF * DIM // 2), jnp.int32),
        ] + [pltpu.SemaphoreType.DMA] * (NBUF + 2),
    )(node_flat, all_idx)

    ids2 = input_ids.reshape(NW, 2 * ROWS_PER_W, HALF).astype(jnp.int32)
    nvocab = tok_emb.shape[0]
    tok_i = pl.pallas_call(
        _pack_body,
        grid=(pl.cdiv(nvocab, 1024),),
        in_specs=[pl.BlockSpec((1024, H), lambda i: (i, 0))],
        out_specs=pl.BlockSpec((1024, H // 2), lambda i: (i, 0)),
        out_shape=jax.ShapeDtypeStruct((nvocab, H // 2), jnp.int32),
    )(tok_emb)
    psum = pl.kernel(
        _lm_pool_body,
        out_type=jax.ShapeDtypeStruct((B, 1, H), f32),
        mesh=mesh,
        scratch_types=[
            pltpu.VMEM((2 * ROWS_PER_W, HALF), jnp.int32),
            pltpu.VMEM((2, HALF, H // 2), jnp.int32),
            pltpu.VMEM((1, H), f32),
            pltpu.SemaphoreType.DMA,
            pltpu.SemaphoreType.DMA,
        ],
    )(tok_i, ids2).reshape(B, H)

    w1a = jnp.kron(jnp.eye(NF, dtype=f32), attW1[:DIM, :])
    w1b = jnp.kron(jnp.eye(NF, dtype=f32), attW1[DIM:, :])
    b1t = jnp.tile(attb1, (NF,))[None, :]
    w2t = jnp.kron(jnp.eye(NF, dtype=f32), attW2)
    b2t = jnp.tile(attb2, (NF,))[None, :]
    exp4 = jnp.kron(jnp.eye(NF, dtype=f32), jnp.ones((1, DIM), f32))

    att = pl.pallas_call(
        _att_body,
        grid=(4, 16),
        in_specs=[
            pl.BlockSpec((2048, 64), lambda lb, j: (1 + 32 * lb + j, 0)),
            pl.BlockSpec((2048, 64), lambda lb, j: (17 + 32 * lb + j, 0)),
            pl.BlockSpec((128, 128), lambda lb, j: (0, 0)),
            pl.BlockSpec((128, 128), lambda lb, j: (0, 0)),
            pl.BlockSpec((1, 128), lambda lb, j: (0, 0)),
            pl.BlockSpec((128, 4), lambda lb, j: (0, 0)),
            pl.BlockSpec((1, 4), lambda lb, j: (0, 0)),
            pl.BlockSpec((4, 128), lambda lb, j: (0, 0)),
        ],
        out_specs=pl.BlockSpec((1, 64, 128), lambda lb, j: (lb, j, 0)),
        out_shape=jax.ShapeDtypeStruct((4, B, 128), f32),
    )(gout_i, gout_i, w1a, w1b, b1t, w2t, b2t, exp4)

    counts = jnp.maximum(jnp.sum(attention_mask, axis=1, keepdims=True), 1).astype(f32)
    linw_pad = jnp.zeros((H, 128), f32).at[:, :2].set(linW)
    linb_pad = jnp.zeros((1, 128), f32).at[0, :2].set(linb)
    g = jnp.arange(H) // 32
    r = jnp.arange(H) % 32
    perm = jnp.where(r < 16, 16 * g + r, H // 2 + 16 * g + (r - 16))
    poolw_perm = poolW[perm, :]
    proba_pad = pl.pallas_call(
        _head_body,
        out_shape=jax.ShapeDtypeStruct((B, 128), f32),
    )(psum, counts, poolw_perm, poolb[None, :], linw_pad, linb_pad)
    proba = proba_pad[:, :2]

    self_uv = _unpack_tc(gout_i[0:2 * B])
    e_u = jnp.stack([self_uv[0:B], att[0], att[1]], axis=1).reshape(B, NL + 1, NF, DIM)
    e_v = jnp.stack([self_uv[B:2 * B], att[2], att[3]], axis=1).reshape(B, NL + 1, NF, DIM)
    return (proba, e_u, e_v)
